# baseline (speedup 1.0000x reference)
import jax
import jax.numpy as jnp
from jax import lax
from jax.experimental import pallas as pl
from jax.experimental.pallas import tpu as pltpu
from jax.experimental.pallas import tpu_sc as plsc

E = 8
N = 8192
D = 1024
L = 16
NC = 2
NS = 16
NW = NC * NS
BW = N // NW
C = 8
NCHUNK = BW // C
NBUF = 4
NSLICE = D // L


def _fuse_body(table, gates, out, gates_v, idx_v, w0_v, w1_v,
               rows, out_v, gsem0, gsem1, gsem2, gsem3,
               osem0, osem1, osem2, osem3):
  wid = lax.axis_index("s") * NC + lax.axis_index("c")
  base = wid * BW

  pltpu.sync_copy(gates.at[pl.ds(base * E, BW * E)], gates_v)

  lanes = lax.iota(jnp.int32, L)

  def route(tb, c):
    tok = tb * L + lanes
    g = [plsc.load_gather(gates_v, [tok * E + e]) for e in range(E)]
    m1 = g[0]
    for e in range(1, E):
      m1 = jnp.maximum(m1, g[e])
    i1 = jnp.full((L,), E, jnp.int32)
    for e in range(E - 1, -1, -1):
      i1 = jnp.where(g[e] == m1, jnp.int32(e), i1)
    neg = jnp.float32(-jnp.inf)
    g2 = [jnp.where(i1 == e, neg, g[e]) for e in range(E)]
    m2 = g2[0]
    for e in range(1, E):
      m2 = jnp.maximum(m2, g2[e])
    i2 = jnp.full((L,), E, jnp.int32)
    for e in range(E - 1, -1, -1):
      i2 = jnp.where(g2[e] == m2, jnp.int32(e), i2)
    s = m1 + m2 + jnp.float32(1e-8)
    off = tb * L
    w0_v[pl.ds(off, L)] = m1 / s
    w1_v[pl.ds(off, L)] = m2 / s
    plsc.store_scatter(idx_v, [tok * 2], i1 * N + base + tok)
    plsc.store_scatter(idx_v, [tok * 2 + 1], i2 * N + base + tok)
    return c

  lax.fori_loop(0, BW // L, route, 0)

  gsems = (gsem0, gsem1, gsem2, gsem3)
  osems = (osem0, osem1, osem2, osem3)

  def issue_gather(cc, b):
    iv = idx_v.at[pl.ds(cc * 2 * C, 2 * C)]
    pltpu.async_copy(table.at[iv], rows.at[b], gsems[b])

  def wait_gather(cc, b):
    iv = idx_v.at[pl.ds(cc * 2 * C, 2 * C)]
    pltpu.make_async_copy(table.at[iv], rows.at[b], gsems[b]).wait()

  def out_slice(cc):
    return out.at[pl.ds(base + cc * C, C)]

  for b in range(NBUF):
    issue_gather(b, b)

  def step(gg, c):
    for b in range(NBUF):
      cc = gg * NBUF + b
      wait_gather(cc, b)

      @pl.when(gg > 0)
      def _():
        pltpu.make_async_copy(out_v.at[b], out_slice(cc - NBUF),
                              osems[b]).wait()

      rb = rows.at[b]
      ob = out_v.at[b]

      def token(t, c2):
        widx = cc * C + t
        w0s = plsc.load_gather(w0_v, [jnp.full((L,), widx, jnp.int32)])
        w1s = plsc.load_gather(w1_v, [jnp.full((L,), widx, jnp.int32)])

        @plsc.parallel_loop(0, NSLICE, unroll=8)
        def _(sidx):
          sl = pl.ds(sidx * L, L)
          ob[t, sl] = w0s * rb[2 * t, sl] + w1s * rb[2 * t + 1, sl]

        return c2

      lax.fori_loop(0, C, token, 0)

      @pl.when(cc + NBUF < NCHUNK)
      def _():
        issue_gather(cc + NBUF, b)

      pltpu.async_copy(ob, out_slice(cc), osems[b])
    return c

  lax.fori_loop(0, NCHUNK // NBUF, step, 0)

  for b in range(NBUF):
    pltpu.make_async_copy(out_v.at[b], out_slice(NCHUNK - NBUF + b),
                          osems[b]).wait()


def kernel(expert_outputs, gate_outputs):
  table = expert_outputs.reshape(E * N, D)
  gates_flat = gate_outputs.reshape(N * E)
  mesh = plsc.VectorSubcoreMesh(core_axis_name="c", subcore_axis_name="s")
  fn = pl.kernel(
      _fuse_body,
      out_type=jax.ShapeDtypeStruct((N, D), jnp.float32),
      mesh=mesh,
      compiler_params=pltpu.CompilerParams(
          needs_layout_passes=False,
          disable_bounds_checks=True,
          disable_semaphore_checks=True,
      ),
      scratch_types=[
          pltpu.VMEM((BW * E,), jnp.float32),
          pltpu.VMEM((2 * BW,), jnp.int32),
          pltpu.VMEM((BW,), jnp.float32),
          pltpu.VMEM((BW,), jnp.float32),
          pltpu.VMEM((NBUF, 2 * C, D), jnp.float32),
          pltpu.VMEM((NBUF, C, D), jnp.float32),
          pltpu.SemaphoreType.DMA,
          pltpu.SemaphoreType.DMA,
          pltpu.SemaphoreType.DMA,
          pltpu.SemaphoreType.DMA,
          pltpu.SemaphoreType.DMA,
          pltpu.SemaphoreType.DMA,
          pltpu.SemaphoreType.DMA,
          pltpu.SemaphoreType.DMA,
      ],
  )
  return fn(table, gates_flat)


# speedup vs baseline: 1.0071x; 1.0071x over previous
import jax
import jax.numpy as jnp
from jax import lax
from jax.experimental import pallas as pl
from jax.experimental.pallas import tpu as pltpu
from jax.experimental.pallas import tpu_sc as plsc

E = 8
N = 8192
D = 1024
L = 16
NC = 2
NS = 16
NW = NC * NS
BW = N // NW
C = 8
NCHUNK = BW // C
NBUF = 4
NSLICE = D // L


def _fuse_body(table, gates, out, gates_v, idx_v, w0_v, w1_v,
               rows, out_v, gsem0, gsem1, gsem2, gsem3,
               osem0, osem1, osem2, osem3):
  wid = lax.axis_index("s") * NC + lax.axis_index("c")
  base = wid * BW

  pltpu.sync_copy(gates.at[pl.ds(base * E, BW * E)], gates_v)

  lanes = lax.iota(jnp.int32, L)

  def route(tb, c):
    tok = tb * L + lanes
    g = [plsc.load_gather(gates_v, [tok * E + e]) for e in range(E)]
    m1 = g[0]
    for e in range(1, E):
      m1 = jnp.maximum(m1, g[e])
    i1 = jnp.full((L,), E, jnp.int32)
    for e in range(E - 1, -1, -1):
      i1 = jnp.where(g[e] == m1, jnp.int32(e), i1)
    neg = jnp.float32(-jnp.inf)
    g2 = [jnp.where(i1 == e, neg, g[e]) for e in range(E)]
    m2 = g2[0]
    for e in range(1, E):
      m2 = jnp.maximum(m2, g2[e])
    i2 = jnp.full((L,), E, jnp.int32)
    for e in range(E - 1, -1, -1):
      i2 = jnp.where(g2[e] == m2, jnp.int32(e), i2)
    s = m1 + m2 + jnp.float32(1e-8)
    off = tb * L
    w0_v[pl.ds(off, L)] = m1 / s
    w1_v[pl.ds(off, L)] = m2 / s
    plsc.store_scatter(idx_v, [tok * 2], i1 * N + base + tok)
    plsc.store_scatter(idx_v, [tok * 2 + 1], i2 * N + base + tok)
    return c

  gsems = (gsem0, gsem1, gsem2, gsem3)
  osems = (osem0, osem1, osem2, osem3)

  def issue_gather(cc, b):
    iv = idx_v.at[pl.ds(cc * 2 * C, 2 * C)]
    pltpu.async_copy(table.at[iv], rows.at[b], gsems[b])

  def wait_gather(cc, b):
    iv = idx_v.at[pl.ds(cc * 2 * C, 2 * C)]
    pltpu.make_async_copy(table.at[iv], rows.at[b], gsems[b]).wait()

  def out_slice(cc):
    return out.at[pl.ds(base + cc * C, C)]

  lax.fori_loop(0, 2, route, 0)
  for b in range(NBUF):
    issue_gather(b, b)
  lax.fori_loop(2, BW // L, route, 0)

  def step(gg, c):
    for b in range(NBUF):
      cc = gg * NBUF + b
      wait_gather(cc, b)

      @pl.when(gg > 0)
      def _():
        pltpu.make_async_copy(out_v.at[b], out_slice(cc - NBUF),
                              osems[b]).wait()

      rb = rows.at[b]
      ob = out_v.at[b]

      def token(t, c2):
        widx = cc * C + t
        w0s = plsc.load_gather(w0_v, [jnp.full((L,), widx, jnp.int32)])
        w1s = plsc.load_gather(w1_v, [jnp.full((L,), widx, jnp.int32)])

        @plsc.parallel_loop(0, NSLICE, unroll=8)
        def _(sidx):
          sl = pl.ds(sidx * L, L)
          ob[t, sl] = w0s * rb[2 * t, sl] + w1s * rb[2 * t + 1, sl]

        return c2

      lax.fori_loop(0, C, token, 0)

      @pl.when(cc + NBUF < NCHUNK)
      def _():
        issue_gather(cc + NBUF, b)

      pltpu.async_copy(ob, out_slice(cc), osems[b])
    return c

  lax.fori_loop(0, NCHUNK // NBUF, step, 0)

  for b in range(NBUF):
    pltpu.make_async_copy(out_v.at[b], out_slice(NCHUNK - NBUF + b),
                          osems[b]).wait()


def kernel(expert_outputs, gate_outputs):
  table = expert_outputs.reshape(E * N, D)
  gates_flat = gate_outputs.reshape(N * E)
  mesh = plsc.VectorSubcoreMesh(core_axis_name="c", subcore_axis_name="s")
  fn = pl.kernel(
      _fuse_body,
      out_type=jax.ShapeDtypeStruct((N, D), jnp.float32),
      mesh=mesh,
      compiler_params=pltpu.CompilerParams(needs_layout_passes=False),
      scratch_types=[
          pltpu.VMEM((BW * E,), jnp.float32),
          pltpu.VMEM((2 * BW,), jnp.int32),
          pltpu.VMEM((BW,), jnp.float32),
          pltpu.VMEM((BW,), jnp.float32),
          pltpu.VMEM((NBUF, 2 * C, D), jnp.float32),
          pltpu.VMEM((NBUF, C, D), jnp.float32),
          pltpu.SemaphoreType.DMA,
          pltpu.SemaphoreType.DMA,
          pltpu.SemaphoreType.DMA,
          pltpu.SemaphoreType.DMA,
          pltpu.SemaphoreType.DMA,
          pltpu.SemaphoreType.DMA,
          pltpu.SemaphoreType.DMA,
          pltpu.SemaphoreType.DMA,
      ],
  )
  return fn(table, gates_flat)
